# baseline (speedup 1.0000x reference)
import functools

import jax
import jax.numpy as jnp
from jax import lax
from jax.experimental import pallas as pl
from jax.experimental.pallas import tpu as pltpu
from jax.experimental.pallas import tpu_sc as plsc

_NC = 2
_NS = 16
_NW = _NC * _NS
_CHUNK = 128

_BM = 1024
_CAT = 128


@functools.partial(jax.jit, static_argnames=("b", "d2"))
def _gather_pairs(table2, idx2d, b, d2):
    bpw = b // _NW
    nch = bpw // _CHUNK
    mesh = plsc.VectorSubcoreMesh(core_axis_name="c", subcore_axis_name="s")

    @functools.partial(
        pl.kernel,
        mesh=mesh,
        out_type=jax.ShapeDtypeStruct((b, d2), jnp.float32),
        scratch_types=[
            pltpu.VMEM((nch, _CHUNK), jnp.int32),
            pltpu.VMEM((bpw, d2), jnp.float32),
            pltpu.SemaphoreType.DMA,
        ],
    )
    def gather(table_hbm, idx_hbm, out_hbm, idx_v, rows_v, sem):
        wid = lax.axis_index("s") * _NC + lax.axis_index("c")
        pltpu.sync_copy(idx_hbm.at[pl.ds(wid * nch, nch)], idx_v)
        copies = []
        for j in range(nch):
            copies.append(pltpu.async_copy(
                table_hbm.at[idx_v.at[j]],
                rows_v.at[pl.ds(j * _CHUNK, _CHUNK)], sem))
        for c in copies:
            c.wait()
        pltpu.sync_copy(rows_v, out_hbm.at[pl.ds(wid * bpw, bpw)])

    return gather(table2, idx2d)


def _mlp_block(uid_ref, gender_ref, age_ref, pair_ref, w1u_ref, ct_ref,
               w1ga_ref, b1_ref, w2_ref, b2_ref, out_ref):
    u = uid_ref[0, 0, :]
    g = gender_ref[0, 0, :]
    a = age_ref[0, 0, :]
    d = w1u_ref.shape[0]
    odd = (u & 1)[:, None] == 1
    ue = jnp.where(odd, pair_ref[:, d:], pair_ref[:, :d])
    iota = lax.broadcasted_iota(jnp.int32, (_BM, _CAT), 1)
    onehot = jnp.logical_or(iota == g[:, None],
                            iota == a[:, None] + 3).astype(jnp.float32)
    ga = jnp.dot(ct_ref[...], w1ga_ref[...], preferred_element_type=jnp.float32)
    h = jnp.dot(ue, w1u_ref[...], preferred_element_type=jnp.float32)
    h = h + jnp.dot(onehot, ga, preferred_element_type=jnp.float32) + b1_ref[...]
    h = jnp.where(h >= 0, h, 0.01 * h)
    out_ref[...] = (jnp.dot(h, w2_ref[...], preferred_element_type=jnp.float32)
                    + b2_ref[...])


def kernel(uid, gender, age, uid_table, gender_table, age_table, W1, b1, W2, b2):
    b = uid.shape[0]
    v_uid, user_dim = uid_table.shape
    g_dim = gender_table.shape[1]
    a_dim = age_table.shape[1]
    model_dim = W2.shape[1]

    uid = uid.astype(jnp.int32)
    table2 = uid_table.reshape(v_uid // 2, 2 * user_dim)
    idx2d = (uid >> 1).reshape(b // _CHUNK, _CHUNK)
    pair_rows = _gather_pairs(table2, idx2d, b, 2 * user_dim)

    ct = jnp.zeros((_CAT, g_dim + a_dim), jnp.float32)
    ct = ct.at[0:gender_table.shape[0], 0:g_dim].set(gender_table)
    ct = ct.at[3:3 + age_table.shape[0], g_dim:].set(age_table)

    nb = b // _BM
    out = pl.pallas_call(
        _mlp_block,
        grid=(nb,),
        in_specs=[
            pl.BlockSpec((1, 1, _BM), lambda i: (i, 0, 0)),
            pl.BlockSpec((1, 1, _BM), lambda i: (i, 0, 0)),
            pl.BlockSpec((1, 1, _BM), lambda i: (i, 0, 0)),
            pl.BlockSpec((_BM, 2 * user_dim), lambda i: (i, 0)),
            pl.BlockSpec((user_dim, model_dim), lambda i: (0, 0)),
            pl.BlockSpec((_CAT, g_dim + a_dim), lambda i: (0, 0)),
            pl.BlockSpec((g_dim + a_dim, model_dim), lambda i: (0, 0)),
            pl.BlockSpec((1, model_dim), lambda i: (0, 0)),
            pl.BlockSpec((model_dim, model_dim), lambda i: (0, 0)),
            pl.BlockSpec((1, model_dim), lambda i: (0, 0)),
        ],
        out_specs=pl.BlockSpec((_BM, model_dim), lambda i: (i, 0)),
        out_shape=jax.ShapeDtypeStruct((b, model_dim), jnp.float32),
        compiler_params=pltpu.CompilerParams(
            dimension_semantics=("arbitrary",)),
    )(uid.reshape(nb, 1, _BM), gender.reshape(nb, 1, _BM),
      age.reshape(nb, 1, _BM), pair_rows,
      W1[:user_dim], ct, W1[user_dim:], b1.reshape(1, model_dim), W2,
      b2.reshape(1, model_dim))
    return out[:, None, :]


# speedup vs baseline: 1.6385x; 1.6385x over previous
import functools

import jax
import jax.numpy as jnp
from jax import lax
from jax.experimental import pallas as pl
from jax.experimental.pallas import tpu as pltpu
from jax.experimental.pallas import tpu_sc as plsc

_NC = 2
_NS = 16
_NW = _NC * _NS

_BM = 1024
_CAT = 128


@functools.partial(jax.jit, static_argnames=("b", "d"))
def _gather_rows(table, idx, b, d):
    bpw = b // _NW
    mesh = plsc.VectorSubcoreMesh(core_axis_name="c", subcore_axis_name="s")

    @functools.partial(
        pl.kernel,
        mesh=mesh,
        out_type=jax.ShapeDtypeStruct((b, d), jnp.float32),
        scratch_types=[
            pltpu.VMEM((bpw,), jnp.int32),
            pltpu.VMEM((bpw, d), jnp.float32),
            pltpu.SemaphoreType.DMA,
        ],
    )
    def gather(table_hbm, idx_hbm, out_hbm, idx_v, rows_v, sem):
        wid = lax.axis_index("s") * _NC + lax.axis_index("c")
        base = wid * bpw
        pltpu.sync_copy(idx_hbm.at[pl.ds(base, bpw)], idx_v)

        def body(gidx, carry):
            vec = idx_v[pl.ds(gidx * 16, 16)]
            for k in range(16):
                pltpu.async_copy(table_hbm.at[vec[k]],
                                 rows_v.at[gidx * 16 + k], sem)
            return carry

        lax.fori_loop(0, bpw // 16, body, 0)
        pltpu.make_async_copy(table_hbm.at[pl.ds(0, bpw)], rows_v, sem).wait()
        pltpu.sync_copy(rows_v, out_hbm.at[pl.ds(base, bpw)])

    return gather(table, idx)


def _mlp_block(gender_ref, age_ref, uid_ref, w1u_ref, ct_ref, w1ga_ref,
               b1_ref, w2_ref, b2_ref, out_ref):
    g = gender_ref[0, 0, :]
    a = age_ref[0, 0, :]
    iota = lax.broadcasted_iota(jnp.int32, (_BM, _CAT), 1)
    onehot = jnp.logical_or(iota == g[:, None],
                            iota == a[:, None] + 3).astype(jnp.float32)
    ga = jnp.dot(ct_ref[...], w1ga_ref[...], preferred_element_type=jnp.float32)
    h = jnp.dot(uid_ref[...], w1u_ref[...], preferred_element_type=jnp.float32)
    h = h + jnp.dot(onehot, ga, preferred_element_type=jnp.float32) + b1_ref[...]
    h = jnp.where(h >= 0, h, 0.01 * h)
    out_ref[...] = (jnp.dot(h, w2_ref[...], preferred_element_type=jnp.float32)
                    + b2_ref[...])


def kernel(uid, gender, age, uid_table, gender_table, age_table, W1, b1, W2, b2):
    b = uid.shape[0]
    v_uid, user_dim = uid_table.shape
    g_dim = gender_table.shape[1]
    a_dim = age_table.shape[1]
    model_dim = W2.shape[1]

    uid_emb = _gather_rows(uid_table, uid.astype(jnp.int32), b, user_dim)

    ct = jnp.zeros((_CAT, g_dim + a_dim), jnp.float32)
    ct = ct.at[0:gender_table.shape[0], 0:g_dim].set(gender_table)
    ct = ct.at[3:3 + age_table.shape[0], g_dim:].set(age_table)

    nb = b // _BM
    out = pl.pallas_call(
        _mlp_block,
        grid=(nb,),
        in_specs=[
            pl.BlockSpec((1, 1, _BM), lambda i: (i, 0, 0)),
            pl.BlockSpec((1, 1, _BM), lambda i: (i, 0, 0)),
            pl.BlockSpec((_BM, user_dim), lambda i: (i, 0)),
            pl.BlockSpec((user_dim, model_dim), lambda i: (0, 0)),
            pl.BlockSpec((_CAT, g_dim + a_dim), lambda i: (0, 0)),
            pl.BlockSpec((g_dim + a_dim, model_dim), lambda i: (0, 0)),
            pl.BlockSpec((1, model_dim), lambda i: (0, 0)),
            pl.BlockSpec((model_dim, model_dim), lambda i: (0, 0)),
            pl.BlockSpec((1, model_dim), lambda i: (0, 0)),
        ],
        out_specs=pl.BlockSpec((_BM, model_dim), lambda i: (i, 0)),
        out_shape=jax.ShapeDtypeStruct((b, model_dim), jnp.float32),
        compiler_params=pltpu.CompilerParams(
            dimension_semantics=("arbitrary",)),
    )(gender.reshape(nb, 1, _BM), age.reshape(nb, 1, _BM), uid_emb,
      W1[:user_dim], ct, W1[user_dim:], b1.reshape(1, model_dim), W2,
      b2.reshape(1, model_dim))
    return out[:, None, :]
